# baseline (speedup 1.0000x reference)
import functools

import jax
import jax.numpy as jnp
from jax import lax
from jax.experimental import pallas as pl
from jax.experimental.pallas import tpu as pltpu
from jax.experimental.pallas import tpu_sc as plsc

N_NODES = 10000
N_EDGES = 160000
D_IN = 256
D_OUT = 256
DH = 128

NC = 2
NS = 16
KB = 128
NB = 80
NB_C = 40
NQ = 8
QR = KB // NQ
EPT = NB * KB
E_PAD = NS * EPT
N_TRASH = 128
ACC_ROWS = N_NODES + N_TRASH
ROWS_PT = 624
ROWS_TAIL = N_NODES - NS * ROWS_PT


def _bn_mm_body(x_ref, g_ref, b_ref, w_ref, hl_ref, hr_ref):
    x = x_ref[...]
    mean = jnp.mean(x, axis=0, keepdims=True)
    var = jnp.mean((x - mean) * (x - mean), axis=0, keepdims=True)
    scale = g_ref[...] * lax.rsqrt(var + 1e-5)
    xn = (x - mean) * scale + b_ref[...]
    h = lax.dot_general(xn, w_ref[...], (((1,), (1,)), ((), ())),
                        preferred_element_type=jnp.float32)
    hl_ref[...] = h[:, :DH]
    hr_ref[...] = h[:, DH:]


def _relu_body(l_ref, r_ref, o_ref):
    o_ref[:, :DH] = jnp.maximum(l_ref[...], 0.0)
    o_ref[:, DH:] = jnp.maximum(r_ref[...], 0.0)


def _sc_body(hl, hr, src3_hbm, dst3_hbm, zrows, outl, outr,
             acc, src3, dst3, rows, sems):
    cid = lax.axis_index("c")
    sid = lax.axis_index("s")

    pltpu.sync_copy(zrows, acc.at[pl.ds(sid * ROWS_PT, ROWS_PT)])

    @pl.when(sid == 0)
    def _():
        pltpu.sync_copy(zrows.at[pl.ds(0, ROWS_TAIL + N_TRASH)],
                        acc.at[pl.ds(NS * ROWS_PT, ROWS_TAIL + N_TRASH)])

    pltpu.sync_copy(dst3_hbm.at[sid], dst3)
    plsc.subcore_barrier()

    def edge_pass(h_hbm):
        def gather_block(j, b):
            for q in range(NQ):
                pltpu.async_copy(
                    h_hbm.at[src3.at[j, pl.ds(q * QR, QR)]],
                    rows.at[b, pl.ds(q * QR, QR)], sems[b])

        def wait_block(j, b):
            pltpu.make_async_copy(h_hbm.at[src3.at[j]], rows.at[b],
                                  sems[b]).wait()

        for c in range(NB // NB_C):
            pltpu.sync_copy(src3_hbm.at[sid, pl.ds(c * NB_C, NB_C)], src3)
            gather_block(0, 0)
            gather_block(1, 1)

            def pair(i, c=c):
                for b in range(2):
                    j = 2 * i + b
                    wait_block(j, b)
                    pltpu.sync_copy(rows.at[b], acc.at[dst3.at[c * NB_C + j]],
                                    add=True)

                    @pl.when(j + 2 < NB_C)
                    def _(j=j, b=b):
                        gather_block(j + 2, b)

            lax.fori_loop(0, NB_C // 2, lambda i, _, c=c: pair(i, c), None)

    @pl.when(cid == 0)
    def _():
        edge_pass(hl)

    @pl.when(cid == 1)
    def _():
        edge_pass(hr)

    plsc.subcore_barrier()

    r0 = sid * ROWS_PT

    @pl.when(cid == 0)
    def _():
        pltpu.sync_copy(acc.at[pl.ds(r0, ROWS_PT)], outl.at[pl.ds(r0, ROWS_PT)])

        @pl.when(sid == 0)
        def _():
            pltpu.sync_copy(acc.at[pl.ds(NS * ROWS_PT, ROWS_TAIL)],
                            outl.at[pl.ds(NS * ROWS_PT, ROWS_TAIL)])

    @pl.when(cid == 1)
    def _():
        pltpu.sync_copy(acc.at[pl.ds(r0, ROWS_PT)], outr.at[pl.ds(r0, ROWS_PT)])

        @pl.when(sid == 0)
        def _():
            pltpu.sync_copy(acc.at[pl.ds(NS * ROWS_PT, ROWS_TAIL)],
                            outr.at[pl.ds(NS * ROWS_PT, ROWS_TAIL)])


_sc_scatter = functools.partial(
    pl.kernel,
    out_type=(
        jax.ShapeDtypeStruct((N_NODES, DH), jnp.float32),
        jax.ShapeDtypeStruct((N_NODES, DH), jnp.float32),
    ),
    mesh=plsc.VectorSubcoreMesh(core_axis_name="c", subcore_axis_name="s",
                                num_cores=NC, num_subcores=NS),
    scratch_types=[
        pltpu.VMEM_SHARED((ACC_ROWS, DH), jnp.float32),
        pltpu.VMEM((NB_C, KB), jnp.int32),
        pltpu.VMEM((NB, KB), jnp.int32),
        pltpu.VMEM((2, KB, DH), jnp.float32),
        [pltpu.SemaphoreType.DMA] * 2,
    ],
)(_sc_body)


@jax.jit
def kernel(x, edge_index, gamma, beta, W):
    hl, hr = pl.pallas_call(
        _bn_mm_body,
        out_shape=(
            jax.ShapeDtypeStruct((N_NODES, DH), jnp.float32),
            jax.ShapeDtypeStruct((N_NODES, DH), jnp.float32),
        ),
    )(x, gamma.reshape(1, D_IN), beta.reshape(1, D_IN), W)

    pad = E_PAD - N_EDGES
    src = jnp.concatenate(
        [edge_index[0],
         jnp.arange(pad, dtype=jnp.int32) % N_NODES]).reshape(NS, NB, KB)
    dst = jnp.concatenate(
        [edge_index[1],
         N_NODES + (jnp.arange(pad, dtype=jnp.int32) % N_TRASH)]
    ).reshape(NS, NB, KB)
    zrows = jnp.zeros((ROWS_PT, DH), jnp.float32)
    outl, outr = _sc_scatter(hl, hr, src, dst, zrows)

    out = pl.pallas_call(
        _relu_body,
        out_shape=jax.ShapeDtypeStruct((N_NODES, D_OUT), jnp.float32),
    )(outl, outr)
    return out


# speedup vs baseline: 1.0501x; 1.0501x over previous
import functools

import jax
import jax.numpy as jnp
from jax import lax
from jax.experimental import pallas as pl
from jax.experimental.pallas import tpu as pltpu
from jax.experimental.pallas import tpu_sc as plsc

N_NODES = 10000
N_EDGES = 160000
D_IN = 256
D_OUT = 256
DH = 128

NC = 2
NS = 16
KB = 128
NB = 80
NB_C = 40
NQ = 8
QR = KB // NQ
EPT = NB * KB
E_PAD = NS * EPT
N_TRASH = 128
ACC_ROWS = N_NODES + N_TRASH
ROWS_PT = 624
ROWS_TAIL = N_NODES - NS * ROWS_PT


def _bn_mm_body(x_ref, g_ref, b_ref, w_ref, hl_ref, hr_ref):
    x = x_ref[...]
    mean = jnp.mean(x, axis=0, keepdims=True)
    var = jnp.mean((x - mean) * (x - mean), axis=0, keepdims=True)
    scale = g_ref[...] * lax.rsqrt(var + 1e-5)
    xn = (x - mean) * scale + b_ref[...]
    h = lax.dot_general(xn, w_ref[...], (((1,), (1,)), ((), ())),
                        preferred_element_type=jnp.float32)
    hl_ref[...] = h[:, :DH]
    hr_ref[...] = h[:, DH:]


def _sc_body(hl, hr, src3_hbm, dst3_hbm, zrows, out,
             acc, src3, dst3, rows, sems):
    cid = lax.axis_index("c")
    sid = lax.axis_index("s")

    pltpu.sync_copy(zrows, acc.at[pl.ds(sid * ROWS_PT, ROWS_PT)])

    @pl.when(sid == 0)
    def _():
        pltpu.sync_copy(zrows.at[pl.ds(0, ROWS_TAIL + N_TRASH)],
                        acc.at[pl.ds(NS * ROWS_PT, ROWS_TAIL + N_TRASH)])

    pltpu.sync_copy(dst3_hbm.at[sid], dst3)
    plsc.subcore_barrier()

    def edge_pass(h_hbm):
        def gather_block(j, b):
            for q in range(NQ):
                pltpu.async_copy(
                    h_hbm.at[src3.at[j, pl.ds(q * QR, QR)]],
                    rows.at[b, pl.ds(q * QR, QR)], sems[b])

        def wait_block(j, b):
            for q in range(NQ):
                pltpu.make_async_copy(
                    h_hbm.at[src3.at[j, pl.ds(q * QR, QR)]],
                    rows.at[b, pl.ds(q * QR, QR)], sems[b]).wait()

        for c in range(NB // NB_C):
            pltpu.sync_copy(src3_hbm.at[sid, pl.ds(c * NB_C, NB_C)], src3)
            gather_block(0, 0)
            gather_block(1, 1)

            def pair(i, c=c):
                for b in range(2):
                    j = 2 * i + b
                    wait_block(j, b)
                    pltpu.sync_copy(rows.at[b], acc.at[dst3.at[c * NB_C + j]],
                                    add=True)

                    @pl.when(j + 2 < NB_C)
                    def _(j=j, b=b):
                        gather_block(j + 2, b)

            lax.fori_loop(0, NB_C // 2, lambda i, _, c=c: pair(i, c), None)

    @pl.when(cid == 0)
    def _():
        edge_pass(hl)

    @pl.when(cid == 1)
    def _():
        edge_pass(hr)

    plsc.subcore_barrier()

    def relu_chunk(r, n):
        buf = rows.at[0]
        pltpu.sync_copy(acc.at[pl.ds(r, n)], buf.at[pl.ds(0, n)])

        def row_relu(k, _):
            for p in range(DH // 16):
                s = buf[k, pl.ds(p * 16, 16)]
                buf[k, pl.ds(p * 16, 16)] = jnp.maximum(s, 0.0)
            return _

        lax.fori_loop(0, n, row_relu, None)

        @pl.when(cid == 0)
        def _():
            pltpu.sync_copy(buf.at[pl.ds(0, n)],
                            out.at[pl.ds(r, n), pl.ds(0, DH)])

        @pl.when(cid == 1)
        def _():
            pltpu.sync_copy(buf.at[pl.ds(0, n)],
                            out.at[pl.ds(r, n), pl.ds(DH, DH)])

    r0 = sid * ROWS_PT
    for o in range(0, 512, 128):
        relu_chunk(r0 + o, 128)
    relu_chunk(r0 + 512, ROWS_PT - 512)

    @pl.when(sid == 0)
    def _():
        relu_chunk(NS * ROWS_PT, ROWS_TAIL)


_sc_scatter = functools.partial(
    pl.kernel,
    out_type=jax.ShapeDtypeStruct((N_NODES, D_OUT), jnp.float32),
    mesh=plsc.VectorSubcoreMesh(core_axis_name="c", subcore_axis_name="s",
                                num_cores=NC, num_subcores=NS),
    scratch_types=[
        pltpu.VMEM_SHARED((ACC_ROWS, DH), jnp.float32),
        pltpu.VMEM((NB_C, KB), jnp.int32),
        pltpu.VMEM((NB, KB), jnp.int32),
        pltpu.VMEM((2, KB, DH), jnp.float32),
        [pltpu.SemaphoreType.DMA] * 2,
    ],
)(_sc_body)


@jax.jit
def kernel(x, edge_index, gamma, beta, W):
    hl, hr = pl.pallas_call(
        _bn_mm_body,
        out_shape=(
            jax.ShapeDtypeStruct((N_NODES, DH), jnp.float32),
            jax.ShapeDtypeStruct((N_NODES, DH), jnp.float32),
        ),
    )(x, gamma.reshape(1, D_IN), beta.reshape(1, D_IN), W)

    pad = E_PAD - N_EDGES
    src = jnp.concatenate(
        [edge_index[0],
         jnp.arange(pad, dtype=jnp.int32) % N_NODES]).reshape(NS, NB, KB)
    dst = jnp.concatenate(
        [edge_index[1],
         N_NODES + (jnp.arange(pad, dtype=jnp.int32) % N_TRASH)]
    ).reshape(NS, NB, KB)
    zrows = jnp.zeros((ROWS_PT, DH), jnp.float32)
    return _sc_scatter(hl, hr, src, dst, zrows)
